# baseline (speedup 1.0000x reference)
import functools

import jax
import jax.numpy as jnp
from jax import lax
from jax.experimental import pallas as pl
from jax.experimental.pallas import tpu as pltpu
from jax.experimental.pallas import tpu_sc as plsc

E = 64
TOP_K = 2
H = 768
I_DIM = 768
T = 2048
ALPHA = 1.702
LIMIT = 7.0

CH = 128
P = 4864

SC_CORES = 2
SC_SUBCORES = 16
SC_LANES = 16


_RB = 128


def _router_body(x_ref, rw_ref, rb_ref, pa_ref, pb_ref, w1_ref, w2_ref,
                 st_ref, en_ref):
    x = x_ref[...]
    rw = rw_ref[...]
    logits = lax.dot_general(x, rw, (((1,), (1,)), ((), ())),
                             preferred_element_type=jnp.float32)
    logits = logits + rb_ref[...][None, :]
    cols = lax.broadcasted_iota(jnp.int32, logits.shape, 1)
    l1 = jnp.max(logits, axis=1, keepdims=True)
    i1 = jnp.min(jnp.where(logits == l1, cols, E), axis=1, keepdims=True)
    oh1 = cols == i1
    masked = jnp.where(oh1, -jnp.inf, logits)
    l2 = jnp.max(masked, axis=1, keepdims=True)
    i2 = jnp.min(jnp.where(masked == l2, cols, E), axis=1, keepdims=True)
    oh2 = cols == i2
    w1 = 1.0 / (1.0 + jnp.exp(l2 - l1))
    w1_ref[...] = w1[:, 0]
    w2_ref[...] = 1.0 - w1[:, 0]

    oh1f = oh1.astype(jnp.float32)
    oh2f = oh2.astype(jnp.float32)
    ohf = oh1f + oh2f
    counts = jnp.sum(ohf, axis=0, keepdims=True)
    pcounts = jnp.ceil(counts / 8.0) * 8.0
    ce = lax.broadcasted_iota(jnp.int32, (E, E), 0)
    re = lax.broadcasted_iota(jnp.int32, (E, E), 1)
    tri_e = jnp.where(ce < re, 1.0, 0.0)
    starts = jnp.dot(pcounts, tri_e,
                     preferred_element_type=jnp.float32)
    st_ref[...] = starts[0].astype(jnp.int32)
    en_ref[...] = (starts[0] + counts[0]).astype(jnp.int32)

    rb_i = lax.broadcasted_iota(jnp.int32, (_RB, _RB), 0)
    cb_i = lax.broadcasted_iota(jnp.int32, (_RB, _RB), 1)
    tri_b = jnp.where(cb_i < rb_i, 1.0, 0.0)
    pref = jnp.zeros((1, E), jnp.float32)
    for b in range(T // _RB):
        sl = slice(b * _RB, (b + 1) * _RB)
        seg = ohf[sl]
        rank = jnp.dot(tri_b, seg,
                       preferred_element_type=jnp.float32) + pref
        pos_a = jnp.sum((starts + rank) * oh1f[sl], axis=1)
        pos_b = jnp.sum((starts + rank) * oh2f[sl], axis=1)
        pa_ref[sl] = pos_a.astype(jnp.int32)
        pb_ref[sl] = pos_b.astype(jnp.int32)
        pref = pref + jnp.sum(seg, axis=0, keepdims=True)


def _router_tc(x, rw, rb):
    return pl.pallas_call(
        _router_body,
        out_shape=(
            jax.ShapeDtypeStruct((T,), jnp.int32),
            jax.ShapeDtypeStruct((T,), jnp.int32),
            jax.ShapeDtypeStruct((T,), jnp.float32),
            jax.ShapeDtypeStruct((T,), jnp.float32),
            jax.ShapeDtypeStruct((E,), jnp.int32),
            jax.ShapeDtypeStruct((E,), jnp.int32),
        ),
    )(x, rw, rb)


def _sc_gather(x, tok):
    nc, ns = SC_CORES, SC_SUBCORES
    nw = nc * ns
    bpw = P // nw
    splits = (40, 40, 40, 32)
    mesh = plsc.VectorSubcoreMesh(core_axis_name="c", subcore_axis_name="s")

    @functools.partial(
        pl.kernel, mesh=mesh,
        out_type=jax.ShapeDtypeStruct((P, H), jnp.float32),
        scratch_types=[
            pltpu.VMEM((s,), jnp.int32) for s in splits
        ] + [
            pltpu.VMEM((s, H), jnp.float32) for s in splits
        ] + [
            pltpu.SemaphoreType.DMA,
            pltpu.SemaphoreType.DMA,
            pltpu.SemaphoreType.DMA,
            pltpu.SemaphoreType.DMA,
            pltpu.SemaphoreType.DMA,
        ],
    )
    def gather(x_hbm, tok_hbm, out_hbm, i0, i1, i2, i3, r0, r1, r2, r3,
               s0, s1, s2, s3, sem_out):
        idx = (i0, i1, i2, i3)
        rows = (r0, r1, r2, r3)
        sems = (s0, s1, s2, s3)
        wid = lax.axis_index("s") * nc + lax.axis_index("c")
        base = wid * bpw
        off = 0
        for k, s in enumerate(splits):
            pltpu.sync_copy(tok_hbm.at[pl.ds(base + off, s)], idx[k])
            off += s
        gets = [pltpu.async_copy(x_hbm.at[idx[k]], rows[k], sems[k])
                for k in range(len(splits))]
        off = 0
        puts = []
        for k, s in enumerate(splits):
            gets[k].wait()
            puts.append(pltpu.async_copy(
                rows[k], out_hbm.at[pl.ds(base + off, s)], sem_out))
            off += s
        for p in puts:
            p.wait()

    return gather(x, tok)


def _mlp_body(starts_ref, ends_ref, xs_ref, wg_ref, bg_ref, wu_ref, bu_ref,
              wd_ref, bd_ref, sc_ref, y_ref):
    e = pl.program_id(0)
    start = pl.multiple_of(starts_ref[e], 8)
    end = ends_ref[e]
    n = (end - start + (CH - 1)) // CH

    def chunk(j, carry):
        base = start + j * CH
        xs = xs_ref[pl.ds(base, CH), :]
        g = jnp.dot(xs, wg_ref[0], preferred_element_type=jnp.float32)
        g = g + bg_ref[0]
        u = jnp.dot(xs, wu_ref[0], preferred_element_type=jnp.float32)
        u = u + bu_ref[0]
        g = jnp.minimum(g, LIMIT)
        u = jnp.clip(u, -LIMIT, LIMIT)
        h = g * (1.0 / (1.0 + jnp.exp(-ALPHA * g))) * (u + 1.0)
        y = jnp.dot(h, wd_ref[0], preferred_element_type=jnp.float32)
        y = (y + bd_ref[0]) * sc_ref[pl.ds(base, CH), :]
        y_ref[pl.ds(base, CH), :] = y
        return carry

    lax.fori_loop(0, n, chunk, 0)


def _mlp_tc(starts, ends, xs, w_gate, b_gate, w_up, b_up, w_down, b_down,
            scale):
    grid_spec = pltpu.PrefetchScalarGridSpec(
        num_scalar_prefetch=2,
        grid=(E,),
        in_specs=[
            pl.BlockSpec((P, H), lambda e, s0, s1: (0, 0)),
            pl.BlockSpec((1, H, I_DIM), lambda e, s0, s1: (e, 0, 0)),
            pl.BlockSpec((1, 1, I_DIM), lambda e, s0, s1: (e, 0, 0)),
            pl.BlockSpec((1, H, I_DIM), lambda e, s0, s1: (e, 0, 0)),
            pl.BlockSpec((1, 1, I_DIM), lambda e, s0, s1: (e, 0, 0)),
            pl.BlockSpec((1, I_DIM, H), lambda e, s0, s1: (e, 0, 0)),
            pl.BlockSpec((1, 1, H), lambda e, s0, s1: (e, 0, 0)),
            pl.BlockSpec((P, 1), lambda e, s0, s1: (0, 0)),
        ],
        out_specs=pl.BlockSpec((P, H), lambda e, s0, s1: (0, 0)),
    )
    return pl.pallas_call(
        _mlp_body,
        grid_spec=grid_spec,
        out_shape=jax.ShapeDtypeStruct((P, H), jnp.float32),
        compiler_params=pltpu.CompilerParams(
            dimension_semantics=("arbitrary",)),
    )(starts, ends, xs, w_gate, b_gate[:, None, :], w_up, b_up[:, None, :],
      w_down, b_down[:, None, :], scale[:, None])


def _sc_combine(ys, pos_a, pos_b):
    nc, ns, lanes = SC_CORES, SC_SUBCORES, SC_LANES
    nw = nc * ns
    bpt = T // nw
    mesh = plsc.VectorSubcoreMesh(core_axis_name="c", subcore_axis_name="s")

    @functools.partial(
        pl.kernel, mesh=mesh,
        out_type=jax.ShapeDtypeStruct((T, H), jnp.float32),
        scratch_types=[
            pltpu.VMEM((bpt,), jnp.int32),
            pltpu.VMEM((bpt,), jnp.int32),
            pltpu.VMEM((bpt, H), jnp.float32),
            pltpu.VMEM((bpt, H), jnp.float32),
            pltpu.SemaphoreType.DMA,
        ],
    )
    def combine(y_hbm, pa_hbm, pb_hbm, out_hbm, pa_v, pb_v, bufa, bufb, sem):
        wid = lax.axis_index("s") * nc + lax.axis_index("c")
        base = wid * bpt
        pltpu.sync_copy(pa_hbm.at[pl.ds(base, bpt)], pa_v)
        pltpu.sync_copy(pb_hbm.at[pl.ds(base, bpt)], pb_v)
        ca = pltpu.async_copy(y_hbm.at[pa_v], bufa, sem)
        cb = pltpu.async_copy(y_hbm.at[pb_v], bufb, sem)
        ca.wait()
        cb.wait()

        @plsc.parallel_loop(0, bpt, unroll=2)
        def row(r):
            for j in range(H // lanes):
                a = bufa[r, pl.ds(j * lanes, lanes)]
                b = bufb[r, pl.ds(j * lanes, lanes)]
                bufa[r, pl.ds(j * lanes, lanes)] = a + b

        pltpu.sync_copy(bufa, out_hbm.at[pl.ds(base, bpt)])

    return combine(ys, pos_a, pos_b)


def kernel(x, router_w, router_b, w_gate, b_gate, w_up, b_up, w_down, b_down):
    pos_a, pos_b, w1, w2, starts, ends = _router_tc(x, router_w, router_b)

    dest = jnp.stack([pos_a, pos_b], axis=1).reshape(-1)
    tok = jnp.arange(T * TOP_K, dtype=jnp.int32) // TOP_K
    wts = jnp.stack([w1, w2], axis=1).reshape(-1)
    tok_pad = jnp.zeros((P,), jnp.int32).at[dest].set(tok)
    scale = jnp.zeros((P,), jnp.float32).at[dest].set(wts)

    xs = _sc_gather(x, tok_pad)
    ys = _mlp_tc(starts, ends, xs, w_gate, b_gate, w_up, b_up, w_down, b_down,
                 scale)
    return _sc_combine(ys, pos_a, pos_b)


# speedup vs baseline: 1.2423x; 1.2423x over previous
import functools

import jax
import jax.numpy as jnp
from jax import lax
from jax.experimental import pallas as pl
from jax.experimental.pallas import tpu as pltpu
from jax.experimental.pallas import tpu_sc as plsc

E = 64
TOP_K = 2
H = 768
I_DIM = 768
T = 2048
ALPHA = 1.702
LIMIT = 7.0

CH = 128
P = 4864

SC_CORES = 2
SC_SUBCORES = 16
SC_LANES = 16


_RB = 128


def _router_body(x_ref, rw_ref, rb_ref, pa_ref, pb_ref, w1_ref, w2_ref,
                 st_ref, en_ref):
    x = x_ref[...]
    rw = rw_ref[...]
    logits = lax.dot_general(x, rw, (((1,), (1,)), ((), ())),
                             preferred_element_type=jnp.float32)
    logits = logits + rb_ref[...][None, :]
    cols = lax.broadcasted_iota(jnp.int32, logits.shape, 1)
    l1 = jnp.max(logits, axis=1, keepdims=True)
    i1 = jnp.min(jnp.where(logits == l1, cols, E), axis=1, keepdims=True)
    oh1 = cols == i1
    masked = jnp.where(oh1, -jnp.inf, logits)
    l2 = jnp.max(masked, axis=1, keepdims=True)
    i2 = jnp.min(jnp.where(masked == l2, cols, E), axis=1, keepdims=True)
    oh2 = cols == i2
    w1 = 1.0 / (1.0 + jnp.exp(l2 - l1))
    w1_ref[...] = w1[:, 0]
    w2_ref[...] = 1.0 - w1[:, 0]

    oh1f = oh1.astype(jnp.float32)
    oh2f = oh2.astype(jnp.float32)
    ohf = oh1f + oh2f
    counts = jnp.sum(ohf, axis=0, keepdims=True)
    pcounts = jnp.ceil(counts / 8.0) * 8.0
    ce = lax.broadcasted_iota(jnp.int32, (E, E), 0)
    re = lax.broadcasted_iota(jnp.int32, (E, E), 1)
    tri_e = jnp.where(ce < re, 1.0, 0.0)
    starts = jnp.dot(pcounts, tri_e,
                     preferred_element_type=jnp.float32)
    st_ref[...] = starts[0].astype(jnp.int32)
    en_ref[...] = (starts[0] + counts[0]).astype(jnp.int32)

    rb_i = lax.broadcasted_iota(jnp.int32, (_RB, _RB), 0)
    cb_i = lax.broadcasted_iota(jnp.int32, (_RB, _RB), 1)
    tri_b = jnp.where(cb_i < rb_i, 1.0, 0.0)
    pref = jnp.zeros((1, E), jnp.float32)
    for b in range(T // _RB):
        sl = slice(b * _RB, (b + 1) * _RB)
        seg = ohf[sl]
        rank = jnp.dot(tri_b, seg,
                       preferred_element_type=jnp.float32) + pref
        pos_a = jnp.sum((starts + rank) * oh1f[sl], axis=1)
        pos_b = jnp.sum((starts + rank) * oh2f[sl], axis=1)
        pa_ref[sl] = pos_a.astype(jnp.int32)
        pb_ref[sl] = pos_b.astype(jnp.int32)
        pref = pref + jnp.sum(seg, axis=0, keepdims=True)


def _router_tc(x, rw, rb):
    return pl.pallas_call(
        _router_body,
        out_shape=(
            jax.ShapeDtypeStruct((T,), jnp.int32),
            jax.ShapeDtypeStruct((T,), jnp.int32),
            jax.ShapeDtypeStruct((T,), jnp.float32),
            jax.ShapeDtypeStruct((T,), jnp.float32),
            jax.ShapeDtypeStruct((E,), jnp.int32),
            jax.ShapeDtypeStruct((E,), jnp.int32),
        ),
    )(x, rw, rb)


def _sc_gather(x, pos_a, pos_b):
    nc, ns = SC_CORES, SC_SUBCORES
    nw = nc * ns
    tpw = T // nw
    mesh = plsc.VectorSubcoreMesh(core_axis_name="c", subcore_axis_name="s")

    @functools.partial(
        pl.kernel, mesh=mesh,
        out_type=jax.ShapeDtypeStruct((P, H), jnp.float32),
        scratch_types=[
            pltpu.VMEM((tpw,), jnp.int32),
            pltpu.VMEM((tpw,), jnp.int32),
            pltpu.VMEM((tpw, H), jnp.float32),
            pltpu.SemaphoreType.DMA,
            pltpu.SemaphoreType.DMA,
        ],
    )
    def scatter_rows(x_hbm, pa_hbm, pb_hbm, out_hbm, ia_v, ib_v, rows,
                     sem_a, sem_b):
        wid = lax.axis_index("s") * nc + lax.axis_index("c")
        base = wid * tpw
        pltpu.sync_copy(pa_hbm.at[pl.ds(base, tpw)], ia_v)
        pltpu.sync_copy(pb_hbm.at[pl.ds(base, tpw)], ib_v)
        pltpu.sync_copy(x_hbm.at[pl.ds(base, tpw)], rows)
        wa = pltpu.async_copy(rows, out_hbm.at[ia_v], sem_a)
        wb = pltpu.async_copy(rows, out_hbm.at[ib_v], sem_b)
        wa.wait()
        wb.wait()

    return scatter_rows(x, pos_a, pos_b)


def _mlp_body(starts_ref, ends_ref, xs_ref, wg_ref, bg_ref, wu_ref, bu_ref,
              wd_ref, bd_ref, sc_ref, y_ref):
    e = pl.program_id(0)
    start = pl.multiple_of(starts_ref[e], 8)
    end = ends_ref[e]
    n = (end - start + (CH - 1)) // CH

    def chunk(j, carry):
        base = start + j * CH
        xs = xs_ref[pl.ds(base, CH), :]
        g = jnp.dot(xs, wg_ref[0], preferred_element_type=jnp.float32)
        g = g + bg_ref[0]
        u = jnp.dot(xs, wu_ref[0], preferred_element_type=jnp.float32)
        u = u + bu_ref[0]
        g = jnp.minimum(g, LIMIT)
        u = jnp.clip(u, -LIMIT, LIMIT)
        h = g * (1.0 / (1.0 + jnp.exp(-ALPHA * g))) * (u + 1.0)
        y = jnp.dot(h, wd_ref[0], preferred_element_type=jnp.float32)
        y = (y + bd_ref[0]) * sc_ref[pl.ds(base, CH), :]
        y_ref[pl.ds(base, CH), :] = y
        return carry

    lax.fori_loop(0, n, chunk, 0)


def _mlp_tc(starts, ends, xs, w_gate, b_gate, w_up, b_up, w_down, b_down,
            scale):
    grid_spec = pltpu.PrefetchScalarGridSpec(
        num_scalar_prefetch=2,
        grid=(E,),
        in_specs=[
            pl.BlockSpec((P, H), lambda e, s0, s1: (0, 0)),
            pl.BlockSpec((1, H, I_DIM), lambda e, s0, s1: (e, 0, 0)),
            pl.BlockSpec((1, 1, I_DIM), lambda e, s0, s1: (e, 0, 0)),
            pl.BlockSpec((1, H, I_DIM), lambda e, s0, s1: (e, 0, 0)),
            pl.BlockSpec((1, 1, I_DIM), lambda e, s0, s1: (e, 0, 0)),
            pl.BlockSpec((1, I_DIM, H), lambda e, s0, s1: (e, 0, 0)),
            pl.BlockSpec((1, 1, H), lambda e, s0, s1: (e, 0, 0)),
            pl.BlockSpec((P, 1), lambda e, s0, s1: (0, 0)),
        ],
        out_specs=pl.BlockSpec((P, H), lambda e, s0, s1: (0, 0)),
    )
    return pl.pallas_call(
        _mlp_body,
        grid_spec=grid_spec,
        out_shape=jax.ShapeDtypeStruct((P, H), jnp.float32),
        compiler_params=pltpu.CompilerParams(
            dimension_semantics=("arbitrary",)),
    )(starts, ends, xs, w_gate, b_gate[:, None, :], w_up, b_up[:, None, :],
      w_down, b_down[:, None, :], scale[:, None])


def _sc_combine(ys, pos_a, pos_b):
    nc, ns, lanes = SC_CORES, SC_SUBCORES, SC_LANES
    nw = nc * ns
    bpt = T // nw
    mesh = plsc.VectorSubcoreMesh(core_axis_name="c", subcore_axis_name="s")

    @functools.partial(
        pl.kernel, mesh=mesh,
        out_type=jax.ShapeDtypeStruct((T, H), jnp.float32),
        scratch_types=[
            pltpu.VMEM((bpt,), jnp.int32),
            pltpu.VMEM((bpt,), jnp.int32),
            pltpu.VMEM((bpt, H), jnp.float32),
            pltpu.VMEM((bpt, H), jnp.float32),
            pltpu.SemaphoreType.DMA,
        ],
    )
    def combine(y_hbm, pa_hbm, pb_hbm, out_hbm, pa_v, pb_v, bufa, bufb, sem):
        wid = lax.axis_index("s") * nc + lax.axis_index("c")
        base = wid * bpt
        pltpu.sync_copy(pa_hbm.at[pl.ds(base, bpt)], pa_v)
        pltpu.sync_copy(pb_hbm.at[pl.ds(base, bpt)], pb_v)
        ca = pltpu.async_copy(y_hbm.at[pa_v], bufa, sem)
        cb = pltpu.async_copy(y_hbm.at[pb_v], bufb, sem)
        ca.wait()
        cb.wait()

        @plsc.parallel_loop(0, bpt, unroll=2)
        def row(r):
            for j in range(H // lanes):
                a = bufa[r, pl.ds(j * lanes, lanes)]
                b = bufb[r, pl.ds(j * lanes, lanes)]
                bufa[r, pl.ds(j * lanes, lanes)] = a + b

        pltpu.sync_copy(bufa, out_hbm.at[pl.ds(base, bpt)])

    return combine(ys, pos_a, pos_b)


def kernel(x, router_w, router_b, w_gate, b_gate, w_up, b_up, w_down, b_down):
    pos_a, pos_b, w1, w2, starts, ends = _router_tc(x, router_w, router_b)

    dest = jnp.stack([pos_a, pos_b], axis=1).reshape(-1)
    wts = jnp.stack([w1, w2], axis=1).reshape(-1)
    scale = jnp.zeros((P,), jnp.float32).at[dest].set(wts)

    xs = _sc_gather(x, pos_a, pos_b)
    ys = _mlp_tc(starts, ends, xs, w_gate, b_gate, w_up, b_up, w_down, b_down,
                 scale)
    return _sc_combine(ys, pos_a, pos_b)
